# baseline (speedup 1.0000x reference)
import functools

import jax
import jax.numpy as jnp
from jax import lax
from jax.experimental import pallas as pl
from jax.experimental.pallas import tpu as pltpu
from jax.experimental.pallas import tpu_sc as plsc

_NC = 2
_NS = 16
_NW = _NC * _NS
_CHUNK = 128
_BLK_E = 512


def _repack(ent_t):
    D, ENT = ent_t.shape
    grid = pl.cdiv(ENT, _BLK_E)

    def body(x_ref, o_ref):
        x = x_ref[...]
        o_ref[...] = jnp.concatenate(
            [x[:, q * 128:(q + 1) * 128].T for q in range(_BLK_E // 128)],
            axis=1)

    return pl.pallas_call(
        body,
        grid=(grid,),
        in_specs=[pl.BlockSpec((D, _BLK_E), lambda i: (0, i))],
        out_specs=pl.BlockSpec((128, 4 * D), lambda i: (i, 0)),
        out_shape=jax.ShapeDtypeStruct((grid * 128, 4 * D), jnp.float32),
    )(ent_t)


def kernel(p_h, p_t, p_r, n_h, n_t, n_r, ent_emb, rel_emb):
    B = p_h.shape[0]
    ENT, D = ent_emb.shape
    REL = rel_emb.shape[0]
    PACK = 128 // D
    bpw = B // _NW
    n_chunks = bpw // _CHUNK

    ent2 = _repack(ent_emb.T)
    rel_f = rel_emb.reshape(REL * D)

    mesh = plsc.VectorSubcoreMesh(
        core_axis_name="c", subcore_axis_name="s",
        num_cores=_NC, num_subcores=_NS)

    out_t = jax.ShapeDtypeStruct((B,), jnp.float32)
    scratch = (
        [pltpu.VMEM((n_chunks, _CHUNK), jnp.int32) for _ in range(6)]
        + [pltpu.VMEM((n_chunks, _CHUNK), jnp.int32) for _ in range(4)]
        + [pltpu.VMEM((_CHUNK, 128), jnp.float32) for _ in range(2)]
        + [pltpu.VMEM((REL * D,), jnp.float32)]
        + [pltpu.VMEM((_CHUNK * 16,), jnp.float32)]
        + [pltpu.VMEM((bpw,), jnp.float32) for _ in range(2)]
        + [pltpu.SemaphoreType.DMA]
    )

    @functools.partial(
        pl.kernel,
        out_type=(out_t, out_t),
        mesh=mesh,
        scratch_types=scratch,
        compiler_params=pltpu.CompilerParams(needs_layout_passes=False),
    )
    def run(ph_h, pt_h, pr_h, nh_h, nt_h, nr_h, ent_h, rel_h,
            po_h, no_h,
            iv0, iv1, iv2, iv3, iv4, iv5,
            gv0, gv1, gv2, gv3,
            hbuf, tbuf, rel_v, dred_v, op_v, on_v, sem):
        wid = lax.axis_index("s") * _NC + lax.axis_index("c")
        base = wid * bpw

        rel_cp = pltpu.async_copy(rel_h, rel_v, sem)

        idx_hbm = [ph_h, pt_h, nh_h, nt_h, pr_h, nr_h]
        idx_v = [iv0, iv1, iv2, iv3, iv4, iv5]
        for ih, iv in zip(idx_hbm, idx_v):
            for k in range(n_chunks):
                pltpu.sync_copy(ih.at[pl.ds(base + k * _CHUNK, _CHUNK)],
                                iv.at[k])

        gidx_v = [gv0, gv1, gv2, gv3]
        for iv, gv in zip(idx_v[:4], gidx_v):
            for k in range(n_chunks):
                for v in range(_CHUNK // 16):
                    e = iv[k, pl.ds(v * 16, 16)]
                    gv[k, pl.ds(v * 16, 16)] = (
                        lax.shift_left(lax.shift_right_logical(e, 9), 7)
                        + (e & 127))

        rel_cp.wait()

        iota16 = lax.iota(jnp.int32, 16)
        mask_slab = PACK - 1

        def do_side(ihv, itv, irv, ghv, gtv, o_ref):
            for k in range(n_chunks):
                h_cp = pltpu.async_copy(ent_h.at[ghv.at[k]], hbuf, sem)
                t_cp = pltpu.async_copy(ent_h.at[gtv.at[k]], tbuf, sem)
                h_cp.wait()
                t_cp.wait()

                def body1(g, carry):
                    eh16 = ihv[k, pl.ds(g * 16, 16)]
                    et16 = itv[k, pl.ds(g * 16, 16)]
                    er16 = irv[k, pl.ds(g * 16, 16)]
                    sh16 = (lax.shift_right_logical(eh16, 7) & mask_slab) * D
                    st16 = (lax.shift_right_logical(et16, 7) & mask_slab) * D
                    ro16 = er16 * D
                    for j in range(16):
                        row = g * 16 + j
                        d = jnp.zeros((16,), jnp.float32)
                        for c in range(D // 16):
                            hv = hbuf[row, pl.ds(sh16[j] + c * 16, 16)]
                            tv = tbuf[row, pl.ds(st16[j] + c * 16, 16)]
                            rv = rel_v[pl.ds(ro16[j] + c * 16, 16)]
                            d = d + jnp.abs(hv + rv - tv)
                        dred_v[pl.ds(row * 16, 16)] = d
                    return carry
                lax.fori_loop(0, _CHUNK // 16, body1, 0)

                def body2(g, carry):
                    base_idx = g * 256 + iota16 * 16
                    acc = jnp.zeros((16,), jnp.float32)
                    for j in range(16):
                        acc = acc + plsc.load_gather(dred_v, [base_idx + j])
                    o_ref[pl.ds(k * _CHUNK + g * 16, 16)] = acc
                    return carry
                lax.fori_loop(0, _CHUNK // 16, body2, 0)

        do_side(iv0, iv1, iv4, gv0, gv1, op_v)
        do_side(iv2, iv3, iv5, gv2, gv3, on_v)

        pltpu.sync_copy(op_v, po_h.at[pl.ds(base, bpw)])
        pltpu.sync_copy(on_v, no_h.at[pl.ds(base, bpw)])

    return run(p_h, p_t, p_r, n_h, n_t, n_r, ent2, rel_f)


# speedup vs baseline: 3.4028x; 3.4028x over previous
import functools

import jax
import jax.numpy as jnp
from jax import lax
from jax.experimental import pallas as pl
from jax.experimental.pallas import tpu as pltpu
from jax.experimental.pallas import tpu_sc as plsc

_NC = 2
_NS = 16
_NW = _NC * _NS
_CHUNK = 128
_BLK_E = 4096


def _repack(ent_t):
    D, ENT = ent_t.shape
    grid = pl.cdiv(ENT, _BLK_E)
    rows_per_blk = _BLK_E // 4

    def body(x_ref, o_ref):
        x = x_ref[...]
        eye = (lax.broadcasted_iota(jnp.int32, (D, D), 0)
               == lax.broadcasted_iota(jnp.int32, (D, D), 1)
               ).astype(jnp.float32)
        xt = lax.dot_general(
            x, eye, dimension_numbers=(((0,), (0,)), ((), ())),
            preferred_element_type=jnp.float32)
        for s5 in range(_BLK_E // 512):
            for q in range(4):
                beg = s5 * 512 + q * 128
                o_ref[pl.ds(s5 * 128, 128), pl.ds(q * D, D)] = (
                    xt[beg:beg + 128, :])

    return pl.pallas_call(
        body,
        grid=(grid,),
        in_specs=[pl.BlockSpec((D, _BLK_E), lambda i: (0, i))],
        out_specs=pl.BlockSpec((rows_per_blk, 4 * D), lambda i: (i, 0)),
        out_shape=jax.ShapeDtypeStruct((grid * rows_per_blk, 4 * D),
                                       jnp.float32),
    )(ent_t)


def kernel(p_h, p_t, p_r, n_h, n_t, n_r, ent_emb, rel_emb):
    B = p_h.shape[0]
    ENT, D = ent_emb.shape
    REL = rel_emb.shape[0]
    PACK = 128 // D
    bpw = B // _NW
    n_chunks = bpw // _CHUNK

    ent2 = _repack(ent_emb.T)
    rel_f = rel_emb.reshape(REL * D)

    mesh = plsc.VectorSubcoreMesh(
        core_axis_name="c", subcore_axis_name="s",
        num_cores=_NC, num_subcores=_NS)

    out_t = jax.ShapeDtypeStruct((B,), jnp.float32)
    scratch = (
        [pltpu.VMEM((n_chunks, _CHUNK), jnp.int32) for _ in range(6)]
        + [pltpu.VMEM((n_chunks, _CHUNK), jnp.int32) for _ in range(4)]
        + [pltpu.VMEM((_CHUNK, 128), jnp.float32) for _ in range(2)]
        + [pltpu.VMEM((REL * D,), jnp.float32)]
        + [pltpu.VMEM((_CHUNK * 16,), jnp.float32)]
        + [pltpu.VMEM((bpw,), jnp.float32) for _ in range(2)]
        + [pltpu.SemaphoreType.DMA]
    )

    @functools.partial(
        pl.kernel,
        out_type=(out_t, out_t),
        mesh=mesh,
        scratch_types=scratch,
        compiler_params=pltpu.CompilerParams(needs_layout_passes=False),
    )
    def run(ph_h, pt_h, pr_h, nh_h, nt_h, nr_h, ent_h, rel_h,
            po_h, no_h,
            iv0, iv1, iv2, iv3, iv4, iv5,
            gv0, gv1, gv2, gv3,
            hbuf, tbuf, rel_v, dred_v, op_v, on_v, sem):
        wid = lax.axis_index("s") * _NC + lax.axis_index("c")
        base = wid * bpw

        rel_cp = pltpu.async_copy(rel_h, rel_v, sem)

        idx_hbm = [ph_h, pt_h, nh_h, nt_h, pr_h, nr_h]
        idx_v = [iv0, iv1, iv2, iv3, iv4, iv5]
        for ih, iv in zip(idx_hbm, idx_v):
            for k in range(n_chunks):
                pltpu.sync_copy(ih.at[pl.ds(base + k * _CHUNK, _CHUNK)],
                                iv.at[k])

        gidx_v = [gv0, gv1, gv2, gv3]
        for iv, gv in zip(idx_v[:4], gidx_v):
            for k in range(n_chunks):
                for v in range(_CHUNK // 16):
                    e = iv[k, pl.ds(v * 16, 16)]
                    gv[k, pl.ds(v * 16, 16)] = (
                        lax.shift_left(lax.shift_right_logical(e, 9), 7)
                        + (e & 127))

        rel_cp.wait()

        iota16 = lax.iota(jnp.int32, 16)
        mask_slab = PACK - 1

        def do_side(ihv, itv, irv, ghv, gtv, o_ref):
            for k in range(n_chunks):
                h_cp = pltpu.async_copy(ent_h.at[ghv.at[k]], hbuf, sem)
                t_cp = pltpu.async_copy(ent_h.at[gtv.at[k]], tbuf, sem)
                h_cp.wait()
                t_cp.wait()

                def body1(g, carry):
                    eh16 = ihv[k, pl.ds(g * 16, 16)]
                    et16 = itv[k, pl.ds(g * 16, 16)]
                    er16 = irv[k, pl.ds(g * 16, 16)]
                    sh16 = (lax.shift_right_logical(eh16, 7) & mask_slab) * D
                    st16 = (lax.shift_right_logical(et16, 7) & mask_slab) * D
                    ro16 = er16 * D
                    for j in range(16):
                        row = g * 16 + j
                        d = jnp.zeros((16,), jnp.float32)
                        for c in range(D // 16):
                            hv = hbuf[row, pl.ds(sh16[j] + c * 16, 16)]
                            tv = tbuf[row, pl.ds(st16[j] + c * 16, 16)]
                            rv = rel_v[pl.ds(ro16[j] + c * 16, 16)]
                            d = d + jnp.abs(hv + rv - tv)
                        dred_v[pl.ds(row * 16, 16)] = d
                    return carry
                lax.fori_loop(0, _CHUNK // 16, body1, 0)

                def body2(g, carry):
                    base_idx = g * 256 + iota16 * 16
                    acc = jnp.zeros((16,), jnp.float32)
                    for j in range(16):
                        acc = acc + plsc.load_gather(dred_v, [base_idx + j])
                    o_ref[pl.ds(k * _CHUNK + g * 16, 16)] = acc
                    return carry
                lax.fori_loop(0, _CHUNK // 16, body2, 0)

        do_side(iv0, iv1, iv4, gv0, gv1, op_v)
        do_side(iv2, iv3, iv5, gv2, gv3, on_v)

        pltpu.sync_copy(op_v, po_h.at[pl.ds(base, bpw)])
        pltpu.sync_copy(on_v, no_h.at[pl.ds(base, bpw)])

    return run(p_h, p_t, p_r, n_h, n_t, n_r, ent2, rel_f)


# speedup vs baseline: 3.7901x; 1.1138x over previous
import functools

import jax
import jax.numpy as jnp
from jax import lax
from jax.experimental import pallas as pl
from jax.experimental.pallas import tpu as pltpu
from jax.experimental.pallas import tpu_sc as plsc

_NC = 2
_NS = 16
_NW = _NC * _NS
_CHUNK = 128
_BLK_E = 4096


def _repack(ent_t):
    D, ENT = ent_t.shape
    grid = pl.cdiv(ENT, _BLK_E)
    rows_per_blk = _BLK_E // 4

    def body(x_ref, o_ref):
        x = x_ref[...]
        iota_h = lax.broadcasted_iota(jnp.int32, (D, 4 * D), 0)
        iota_l = lax.broadcasted_iota(jnp.int32, (D, 4 * D), 1)
        acc = None
        for q in range(4):
            xq = jnp.concatenate(
                [x[:, 512 * s5 + 128 * q: 512 * s5 + 128 * q + 128]
                 for s5 in range(_BLK_E // 512)], axis=1)
            eq = (iota_l == iota_h + D * q).astype(jnp.float32)
            t = lax.dot_general(
                xq, eq, dimension_numbers=(((0,), (0,)), ((), ())),
                preferred_element_type=jnp.float32)
            acc = t if acc is None else acc + t
        o_ref[...] = acc

    return pl.pallas_call(
        body,
        grid=(grid,),
        in_specs=[pl.BlockSpec((D, _BLK_E), lambda i: (0, i))],
        out_specs=pl.BlockSpec((rows_per_blk, 4 * D), lambda i: (i, 0)),
        out_shape=jax.ShapeDtypeStruct((grid * rows_per_blk, 4 * D),
                                       jnp.float32),
    )(ent_t)


def kernel(p_h, p_t, p_r, n_h, n_t, n_r, ent_emb, rel_emb):
    B = p_h.shape[0]
    ENT, D = ent_emb.shape
    REL = rel_emb.shape[0]
    PACK = 128 // D
    bpw = B // _NW
    n_chunks = bpw // _CHUNK

    ent2 = _repack(ent_emb.T)
    rel_f = rel_emb.reshape(REL * D)

    mesh = plsc.VectorSubcoreMesh(
        core_axis_name="c", subcore_axis_name="s",
        num_cores=_NC, num_subcores=_NS)

    out_t = jax.ShapeDtypeStruct((B,), jnp.float32)
    scratch = (
        [pltpu.VMEM((n_chunks, _CHUNK), jnp.int32) for _ in range(6)]
        + [pltpu.VMEM((n_chunks, _CHUNK), jnp.int32) for _ in range(4)]
        + [pltpu.VMEM((_CHUNK, 128), jnp.float32) for _ in range(2)]
        + [pltpu.VMEM((REL * D,), jnp.float32)]
        + [pltpu.VMEM((_CHUNK * 16,), jnp.float32)]
        + [pltpu.VMEM((bpw,), jnp.float32) for _ in range(2)]
        + [pltpu.SemaphoreType.DMA]
    )

    @functools.partial(
        pl.kernel,
        out_type=(out_t, out_t),
        mesh=mesh,
        scratch_types=scratch,
        compiler_params=pltpu.CompilerParams(needs_layout_passes=False),
    )
    def run(ph_h, pt_h, pr_h, nh_h, nt_h, nr_h, ent_h, rel_h,
            po_h, no_h,
            iv0, iv1, iv2, iv3, iv4, iv5,
            gv0, gv1, gv2, gv3,
            hbuf, tbuf, rel_v, dred_v, op_v, on_v, sem):
        wid = lax.axis_index("s") * _NC + lax.axis_index("c")
        base = wid * bpw

        rel_cp = pltpu.async_copy(rel_h, rel_v, sem)

        idx_hbm = [ph_h, pt_h, nh_h, nt_h, pr_h, nr_h]
        idx_v = [iv0, iv1, iv2, iv3, iv4, iv5]
        for ih, iv in zip(idx_hbm, idx_v):
            for k in range(n_chunks):
                pltpu.sync_copy(ih.at[pl.ds(base + k * _CHUNK, _CHUNK)],
                                iv.at[k])

        gidx_v = [gv0, gv1, gv2, gv3]
        for iv, gv in zip(idx_v[:4], gidx_v):
            for k in range(n_chunks):
                for v in range(_CHUNK // 16):
                    e = iv[k, pl.ds(v * 16, 16)]
                    gv[k, pl.ds(v * 16, 16)] = (
                        lax.shift_left(lax.shift_right_logical(e, 9), 7)
                        + (e & 127))

        rel_cp.wait()

        iota16 = lax.iota(jnp.int32, 16)
        mask_slab = PACK - 1

        def do_side(ihv, itv, irv, ghv, gtv, o_ref):
            for k in range(n_chunks):
                h_cp = pltpu.async_copy(ent_h.at[ghv.at[k]], hbuf, sem)
                t_cp = pltpu.async_copy(ent_h.at[gtv.at[k]], tbuf, sem)
                h_cp.wait()
                t_cp.wait()

                def body1(g, carry):
                    eh16 = ihv[k, pl.ds(g * 16, 16)]
                    et16 = itv[k, pl.ds(g * 16, 16)]
                    er16 = irv[k, pl.ds(g * 16, 16)]
                    sh16 = (lax.shift_right_logical(eh16, 7) & mask_slab) * D
                    st16 = (lax.shift_right_logical(et16, 7) & mask_slab) * D
                    ro16 = er16 * D
                    for j in range(16):
                        row = g * 16 + j
                        d = jnp.zeros((16,), jnp.float32)
                        for c in range(D // 16):
                            hv = hbuf[row, pl.ds(sh16[j] + c * 16, 16)]
                            tv = tbuf[row, pl.ds(st16[j] + c * 16, 16)]
                            rv = rel_v[pl.ds(ro16[j] + c * 16, 16)]
                            d = d + jnp.abs(hv + rv - tv)
                        dred_v[pl.ds(row * 16, 16)] = d
                    return carry
                lax.fori_loop(0, _CHUNK // 16, body1, 0)

                def body2(g, carry):
                    base_idx = g * 256 + iota16 * 16
                    acc = jnp.zeros((16,), jnp.float32)
                    for j in range(16):
                        acc = acc + plsc.load_gather(dred_v, [base_idx + j])
                    o_ref[pl.ds(k * _CHUNK + g * 16, 16)] = acc
                    return carry
                lax.fori_loop(0, _CHUNK // 16, body2, 0)

        do_side(iv0, iv1, iv4, gv0, gv1, op_v)
        do_side(iv2, iv3, iv5, gv2, gv3, on_v)

        pltpu.sync_copy(op_v, po_h.at[pl.ds(base, bpw)])
        pltpu.sync_copy(on_v, no_h.at[pl.ds(base, bpw)])

    return run(p_h, p_t, p_r, n_h, n_t, n_r, ent2, rel_f)


# speedup vs baseline: 4.5244x; 1.1938x over previous
import functools

import jax
import jax.numpy as jnp
from jax import lax
from jax.experimental import pallas as pl
from jax.experimental.pallas import tpu as pltpu
from jax.experimental.pallas import tpu_sc as plsc

_NC = 2
_NS = 16
_NW = _NC * _NS
_CHUNK = 128
_BLK_E = 4096


def _repack(ent_t):
    D, ENT = ent_t.shape
    grid = pl.cdiv(ENT, _BLK_E)
    rows_per_blk = _BLK_E // 4

    def body(x_ref, o_ref):
        x = x_ref[...]
        xbig = jnp.concatenate(
            [jnp.concatenate(
                [x[:, 512 * s5 + 128 * q: 512 * s5 + 128 * q + 128]
                 for s5 in range(_BLK_E // 512)], axis=1)
             for q in range(4)], axis=0)
        eye = (lax.broadcasted_iota(jnp.int32, (4 * D, 4 * D), 0)
               == lax.broadcasted_iota(jnp.int32, (4 * D, 4 * D), 1)
               ).astype(jnp.float32)
        o_ref[...] = lax.dot_general(
            xbig, eye, dimension_numbers=(((0,), (0,)), ((), ())),
            preferred_element_type=jnp.float32)

    return pl.pallas_call(
        body,
        grid=(grid,),
        in_specs=[pl.BlockSpec((D, _BLK_E), lambda i: (0, i))],
        out_specs=pl.BlockSpec((rows_per_blk, 4 * D), lambda i: (i, 0)),
        out_shape=jax.ShapeDtypeStruct((grid * rows_per_blk, 4 * D),
                                       jnp.float32),
    )(ent_t)


def kernel(p_h, p_t, p_r, n_h, n_t, n_r, ent_emb, rel_emb):
    B = p_h.shape[0]
    ENT, D = ent_emb.shape
    REL = rel_emb.shape[0]
    PACK = 128 // D
    bpw = B // _NW
    n_chunks = bpw // _CHUNK

    ent2 = _repack(ent_emb.T)
    rel_f = rel_emb.reshape(REL * D)

    mesh = plsc.VectorSubcoreMesh(
        core_axis_name="c", subcore_axis_name="s",
        num_cores=_NC, num_subcores=_NS)

    out_t = jax.ShapeDtypeStruct((B,), jnp.float32)
    scratch = (
        [pltpu.VMEM((n_chunks, _CHUNK), jnp.int32) for _ in range(6)]
        + [pltpu.VMEM((n_chunks, _CHUNK), jnp.int32) for _ in range(4)]
        + [pltpu.VMEM((_CHUNK, 128), jnp.float32) for _ in range(2)]
        + [pltpu.VMEM((REL * D,), jnp.float32)]
        + [pltpu.VMEM((_CHUNK * 16,), jnp.float32)]
        + [pltpu.VMEM((bpw,), jnp.float32) for _ in range(2)]
        + [pltpu.SemaphoreType.DMA]
    )

    @functools.partial(
        pl.kernel,
        out_type=(out_t, out_t),
        mesh=mesh,
        scratch_types=scratch,
        compiler_params=pltpu.CompilerParams(needs_layout_passes=False),
    )
    def run(ph_h, pt_h, pr_h, nh_h, nt_h, nr_h, ent_h, rel_h,
            po_h, no_h,
            iv0, iv1, iv2, iv3, iv4, iv5,
            gv0, gv1, gv2, gv3,
            hbuf, tbuf, rel_v, dred_v, op_v, on_v, sem):
        wid = lax.axis_index("s") * _NC + lax.axis_index("c")
        base = wid * bpw

        rel_cp = pltpu.async_copy(rel_h, rel_v, sem)

        idx_hbm = [ph_h, pt_h, nh_h, nt_h, pr_h, nr_h]
        idx_v = [iv0, iv1, iv2, iv3, iv4, iv5]
        for ih, iv in zip(idx_hbm, idx_v):
            for k in range(n_chunks):
                pltpu.sync_copy(ih.at[pl.ds(base + k * _CHUNK, _CHUNK)],
                                iv.at[k])

        gidx_v = [gv0, gv1, gv2, gv3]
        for iv, gv in zip(idx_v[:4], gidx_v):
            for k in range(n_chunks):
                for v in range(_CHUNK // 16):
                    e = iv[k, pl.ds(v * 16, 16)]
                    gv[k, pl.ds(v * 16, 16)] = (
                        lax.shift_left(lax.shift_right_logical(e, 9), 7)
                        + (e & 127))

        rel_cp.wait()

        iota16 = lax.iota(jnp.int32, 16)
        mask_slab = PACK - 1

        def do_side(ihv, itv, irv, ghv, gtv, o_ref):
            for k in range(n_chunks):
                h_cp = pltpu.async_copy(ent_h.at[ghv.at[k]], hbuf, sem)
                t_cp = pltpu.async_copy(ent_h.at[gtv.at[k]], tbuf, sem)
                h_cp.wait()
                t_cp.wait()

                def body1(g, carry):
                    eh16 = ihv[k, pl.ds(g * 16, 16)]
                    et16 = itv[k, pl.ds(g * 16, 16)]
                    er16 = irv[k, pl.ds(g * 16, 16)]
                    sh16 = (lax.shift_right_logical(eh16, 7) & mask_slab) * D
                    st16 = (lax.shift_right_logical(et16, 7) & mask_slab) * D
                    ro16 = er16 * D
                    for j in range(16):
                        row = g * 16 + j
                        d = jnp.zeros((16,), jnp.float32)
                        for c in range(D // 16):
                            hv = hbuf[row, pl.ds(sh16[j] + c * 16, 16)]
                            tv = tbuf[row, pl.ds(st16[j] + c * 16, 16)]
                            rv = rel_v[pl.ds(ro16[j] + c * 16, 16)]
                            d = d + jnp.abs(hv + rv - tv)
                        dred_v[pl.ds(row * 16, 16)] = d
                    return carry
                lax.fori_loop(0, _CHUNK // 16, body1, 0)

                def body2(g, carry):
                    base_idx = g * 256 + iota16 * 16
                    acc = jnp.zeros((16,), jnp.float32)
                    for j in range(16):
                        acc = acc + plsc.load_gather(dred_v, [base_idx + j])
                    o_ref[pl.ds(k * _CHUNK + g * 16, 16)] = acc
                    return carry
                lax.fori_loop(0, _CHUNK // 16, body2, 0)

        do_side(iv0, iv1, iv4, gv0, gv1, op_v)
        do_side(iv2, iv3, iv5, gv2, gv3, on_v)

        pltpu.sync_copy(op_v, po_h.at[pl.ds(base, bpw)])
        pltpu.sync_copy(on_v, no_h.at[pl.ds(base, bpw)])

    return run(p_h, p_t, p_r, n_h, n_t, n_r, ent2, rel_f)


# speedup vs baseline: 6.6440x; 1.4685x over previous
import functools

import jax
import jax.numpy as jnp
from jax import lax
from jax.experimental import pallas as pl
from jax.experimental.pallas import tpu as pltpu
from jax.experimental.pallas import tpu_sc as plsc

_NC = 2
_NS = 16
_NW = _NC * _NS
_CHUNK = 128
_BLK_E = 8192


def _repack(ent_t):
    D, ENT = ent_t.shape
    grid = pl.cdiv(ENT, _BLK_E)
    rows_per_blk = _BLK_E // 4

    def body(x_ref, o_ref):
        x = x_ref[...]
        xbig = jnp.concatenate(
            [jnp.concatenate(
                [x[:, 512 * s5 + 128 * q: 512 * s5 + 128 * q + 128]
                 for s5 in range(_BLK_E // 512)], axis=1)
             for q in range(4)], axis=0)
        eye = (lax.broadcasted_iota(jnp.int32, (4 * D, 4 * D), 0)
               == lax.broadcasted_iota(jnp.int32, (4 * D, 4 * D), 1)
               ).astype(jnp.float32)
        o_ref[...] = lax.dot_general(
            xbig, eye, dimension_numbers=(((0,), (0,)), ((), ())),
            preferred_element_type=jnp.float32)

    return pl.pallas_call(
        body,
        grid=(grid,),
        in_specs=[pl.BlockSpec((D, _BLK_E), lambda i: (0, i))],
        out_specs=pl.BlockSpec((rows_per_blk, 4 * D), lambda i: (i, 0)),
        out_shape=jax.ShapeDtypeStruct((grid * rows_per_blk, 4 * D),
                                       jnp.float32),
    )(ent_t)


def kernel(p_h, p_t, p_r, n_h, n_t, n_r, ent_emb, rel_emb):
    B = p_h.shape[0]
    ENT, D = ent_emb.shape
    REL = rel_emb.shape[0]
    PACK = 128 // D
    bpw = B // _NW
    n_chunks = bpw // _CHUNK

    ent2 = _repack(ent_emb.T)
    rel_f = rel_emb.reshape(REL * D)

    mesh = plsc.VectorSubcoreMesh(
        core_axis_name="c", subcore_axis_name="s",
        num_cores=_NC, num_subcores=_NS)

    out_t = jax.ShapeDtypeStruct((B,), jnp.float32)
    scratch = (
        [pltpu.VMEM((n_chunks, _CHUNK), jnp.int32) for _ in range(6)]
        + [pltpu.VMEM((n_chunks, _CHUNK), jnp.int32) for _ in range(4)]
        + [pltpu.VMEM((_CHUNK, 128), jnp.float32) for _ in range(4)]
        + [pltpu.VMEM((REL * D,), jnp.float32)]
        + [pltpu.VMEM((_CHUNK * 16,), jnp.float32)]
        + [pltpu.VMEM((bpw,), jnp.float32) for _ in range(2)]
        + [pltpu.SemaphoreType.DMA, pltpu.SemaphoreType.DMA,
           pltpu.SemaphoreType.DMA]
    )

    @functools.partial(
        pl.kernel,
        out_type=(out_t, out_t),
        mesh=mesh,
        scratch_types=scratch,
        compiler_params=pltpu.CompilerParams(needs_layout_passes=False),
    )
    def run(ph_h, pt_h, pr_h, nh_h, nt_h, nr_h, ent_h, rel_h,
            po_h, no_h,
            iv0, iv1, iv2, iv3, iv4, iv5,
            gv0, gv1, gv2, gv3,
            hb0, tb0, hb1, tb1,
            rel_v, dred_v, op_v, on_v, sem0, sem1, rsem):
        wid = lax.axis_index("s") * _NC + lax.axis_index("c")
        base = wid * bpw

        rel_cp = pltpu.async_copy(rel_h, rel_v, rsem)
        idx_hbm = [ph_h, pt_h, nh_h, nt_h, pr_h, nr_h]
        idx_v = [iv0, iv1, iv2, iv3, iv4, iv5]
        idx_cps = []
        for ih, iv in zip(idx_hbm, idx_v):
            for k in range(n_chunks):
                idx_cps.append(pltpu.async_copy(
                    ih.at[pl.ds(base + k * _CHUNK, _CHUNK)], iv.at[k], rsem))
        for cp in idx_cps:
            cp.wait()

        gidx_v = [gv0, gv1, gv2, gv3]
        for iv, gv in zip(idx_v[:4], gidx_v):
            for k in range(n_chunks):
                for v in range(_CHUNK // 16):
                    e = iv[k, pl.ds(v * 16, 16)]
                    gv[k, pl.ds(v * 16, 16)] = (
                        lax.shift_left(lax.shift_right_logical(e, 9), 7)
                        + (e & 127))

        rel_cp.wait()

        iota16 = lax.iota(jnp.int32, 16)
        mask_slab = PACK - 1
        hbufs = [hb0, hb1]
        tbufs = [tb0, tb1]
        sems = [sem0, sem1]

        phases = []
        for side in range(2):
            ihv, itv = (iv0, iv1) if side == 0 else (iv2, iv3)
            irv = iv4 if side == 0 else iv5
            ghv, gtv = (gv0, gv1) if side == 0 else (gv2, gv3)
            o_ref = op_v if side == 0 else on_v
            for k in range(n_chunks):
                phases.append((ihv, itv, irv, ghv, gtv, o_ref, k))

        def fire(i):
            _, _, _, ghv, gtv, _, k = phases[i]
            b = i % 2
            h_cp = pltpu.async_copy(ent_h.at[ghv.at[k]], hbufs[b], sems[b])
            t_cp = pltpu.async_copy(ent_h.at[gtv.at[k]], tbufs[b], sems[b])
            return (h_cp, t_cp)

        def compute(i):
            ihv, itv, irv, _, _, o_ref, k = phases[i]
            b = i % 2
            hbuf, tbuf = hbufs[b], tbufs[b]

            def body1(g, carry):
                eh16 = ihv[k, pl.ds(g * 16, 16)]
                et16 = itv[k, pl.ds(g * 16, 16)]
                er16 = irv[k, pl.ds(g * 16, 16)]
                sh16 = (lax.shift_right_logical(eh16, 7) & mask_slab) * D
                st16 = (lax.shift_right_logical(et16, 7) & mask_slab) * D
                ro16 = er16 * D
                for j in range(16):
                    row = g * 16 + j
                    d = jnp.zeros((16,), jnp.float32)
                    for c in range(D // 16):
                        hv = hbuf[row, pl.ds(sh16[j] + c * 16, 16)]
                        tv = tbuf[row, pl.ds(st16[j] + c * 16, 16)]
                        rv = rel_v[pl.ds(ro16[j] + c * 16, 16)]
                        d = d + jnp.abs(hv + rv - tv)
                    dred_v[pl.ds(row * 16, 16)] = d
                return carry
            lax.fori_loop(0, _CHUNK // 16, body1, 0)

            def body2(g, carry):
                base_idx = g * 256 + iota16 * 16
                acc = jnp.zeros((16,), jnp.float32)
                for j in range(16):
                    acc = acc + plsc.load_gather(dred_v, [base_idx + j])
                o_ref[pl.ds(k * _CHUNK + g * 16, 16)] = acc
                return carry
            lax.fori_loop(0, _CHUNK // 16, body2, 0)

        inflight = fire(0)
        for i in range(len(phases)):
            nxt = fire(i + 1) if i + 1 < len(phases) else None
            inflight[0].wait()
            inflight[1].wait()
            compute(i)
            inflight = nxt

        pltpu.sync_copy(op_v, po_h.at[pl.ds(base, bpw)])
        pltpu.sync_copy(on_v, no_h.at[pl.ds(base, bpw)])

    return run(p_h, p_t, p_r, n_h, n_t, n_r, ent2, rel_f)
